# baseline (speedup 1.0000x reference)
import jax
import jax.numpy as jnp
from jax import lax
from jax.experimental import pallas as pl
from jax.experimental.pallas import tpu as pltpu

NN = 30
NP = 32
IN = 4
HID = 64
EH = 96
LAT = 16
NMP = 3
ALPHA = 0.2

BB = 16


def _lrelu(v):
    return jnp.maximum(v, ALPHA * v)


def _dott(a, w):
    return lax.dot_general(a, w, (((1,), (1,)), ((), ())))


def _encoder_body(x_ref, ew0_refs, eb0_refs, ew1_refs, eb1_refs,
                  nw0_refs, nb0_refs, nw1_refs, nb1_refs,
                  wout_ref, bout_ref, out_ref):
    B = x_ref.shape[0]
    M = B * NP
    E = B * NP * NP
    h2 = x_ref[...].reshape(M, HID)
    selj = (lax.broadcasted_iota(jnp.int32, (NP * NP, NP), 0) // NP
            == lax.broadcasted_iota(jnp.int32, (NP * NP, NP), 1)
            ).astype(jnp.float32)
    for r in range(NMP):
        ew0 = ew0_refs[r][...]
        w0a = ew0[:, :HID]
        w0b = ew0[:, HID:2 * HID]
        w0d = ew0[:, 2 * HID:].reshape(1, EH)
        b0 = eb0_refs[r][...]
        w1 = ew1_refs[r][...]
        b1 = eb1_refs[r][...]
        nw0 = nw0_refs[r][...]
        nwh = nw0[:, :HID]
        nwa = nw0[:, HID:]
        nb0 = nb0_refs[r][...]
        nw1 = nw1_refs[r][...]
        nb1 = nb1_refs[r][...]

        n2 = jnp.sum(h2 * h2, axis=1, keepdims=True)
        P2 = _dott(h2, w0a) + n2 * w0d + b0
        Q2 = _dott(h2, w0b) + n2 * w0d
        h3 = h2.reshape(B, NP, HID)
        G3 = lax.dot_general(h3, h3, (((2,), (2,)), ((0,), (0,))),
                             precision=lax.Precision.HIGHEST)

        ppart = pltpu.repeat(P2.reshape(B, NP, EH), NP, axis=1)
        q3 = Q2.reshape(B, NP, EH)
        qpart = jnp.concatenate(
            [jnp.dot(selj, q3[b]) for b in range(B)], axis=0)
        r2 = pltpu.repeat(G3, NP, axis=1)
        gl = (r2 * selj[None]).reshape(E, NP)
        w2 = jnp.broadcast_to(-2.0 * w0d, (NP, EH))
        gpart = jnp.dot(gl, w2)

        pre = ppart.reshape(E, EH) + qpart + gpart
        e2 = _lrelu(pre)
        f2 = _lrelu(_dott(e2, w1) + b1)
        f4 = f2.reshape(B, NP, NP, EH)
        agg = jnp.sum(f4[:, :NN], axis=1).reshape(M, EH)
        n1 = _lrelu(_dott(h2, nwh) + _dott(agg, nwa) + nb0)
        h2 = _lrelu(_dott(n1, nw1) + nb1)

    lat = _dott(h2, wout_ref[...]) + bout_ref[...]
    out_ref[...] = lat.reshape(B, NP, LAT)[:, :NN, :]


def _mix_body(lat_ref, wm_ref, out_ref):
    out_ref[...] = _dott(lat_ref[...], wm_ref[...])


def kernel(x, params):
    bs = x.shape[0]
    h0 = jnp.pad(x, ((0, 0), (0, NP - NN), (0, HID - IN)))

    ew0 = [params['ew%d_0' % i] for i in range(NMP)]
    eb0 = [params['eb%d_0' % i][None] for i in range(NMP)]
    ew1 = [params['ew%d_1' % i] for i in range(NMP)]
    eb1 = [params['eb%d_1' % i][None] for i in range(NMP)]
    nw0 = [params['nw%d_0' % i] for i in range(NMP)]
    nb0 = [params['nb%d_0' % i][None] for i in range(NMP)]
    nw1 = [params['nw%d_1' % i] for i in range(NMP)]
    nb1 = [params['nb%d_1' % i][None] for i in range(NMP)]

    def body(x_r, *ws):
        ew0_r = ws[0:3]
        eb0_r = ws[3:6]
        ew1_r = ws[6:9]
        eb1_r = ws[9:12]
        nw0_r = ws[12:15]
        nb0_r = ws[15:18]
        nw1_r = ws[18:21]
        nb1_r = ws[21:24]
        wout_r = ws[24]
        bout_r = ws[25]
        out_r = ws[26]
        _encoder_body(x_r, ew0_r, eb0_r, ew1_r, eb1_r,
                      nw0_r, nb0_r, nw1_r, nb1_r, wout_r, bout_r, out_r)

    full = lambda a: pl.BlockSpec(a.shape, lambda i: (0,) * a.ndim)
    wargs = (*ew0, *eb0, *ew1, *eb1, *nw0, *nb0, *nw1, *nb1,
             params['w_out'], params['b_out'][None])
    lat = pl.pallas_call(
        body,
        grid=(bs // BB,),
        in_specs=[pl.BlockSpec((BB, NP, HID), lambda i: (i, 0, 0))]
                 + [full(a) for a in wargs],
        out_specs=pl.BlockSpec((BB, NN, LAT), lambda i: (i, 0, 0)),
        out_shape=jax.ShapeDtypeStruct((bs, NN, LAT), jnp.float32),
        compiler_params=pltpu.CompilerParams(
            dimension_semantics=("parallel",),
        ),
    )(h0, *wargs)

    z = pl.pallas_call(
        _mix_body,
        out_shape=jax.ShapeDtypeStruct((bs, LAT), jnp.float32),
    )(lat.reshape(bs, NN * LAT), params['w_mix'])
    return z[None]


# speedup vs baseline: 1.0040x; 1.0040x over previous
import jax
import jax.numpy as jnp
from jax import lax
from jax.experimental import pallas as pl
from jax.experimental.pallas import tpu as pltpu

NN = 30
NP = 32
IN = 4
HID = 64
EH = 96
LAT = 16
NMP = 3
ALPHA = 0.2

BB = 16


def _lrelu(v):
    return jnp.maximum(v, ALPHA * v)


def _encoder_body(x_ref, w0a_ref, w0b_ref, w0d_ref, b0_ref, w1_ref, b1_ref,
                  nwh_ref, nwa_ref, nb0_ref, nw1_ref, nb1_ref,
                  wout_ref, bout_ref, out_ref):
    B = x_ref.shape[0]
    M = B * NP
    E = B * NP * NP
    h2 = x_ref[...].reshape(M, HID)
    selj = (lax.broadcasted_iota(jnp.int32, (NP * NP, NP), 0) // NP
            == lax.broadcasted_iota(jnp.int32, (NP * NP, NP), 1)
            ).astype(jnp.float32)
    eye = (lax.broadcasted_iota(jnp.int32, (NP, NP), 0)
           == lax.broadcasted_iota(jnp.int32, (NP, NP), 1)
           ).astype(jnp.float32)
    for r in range(NMP):
        w0a = w0a_ref[r]
        w0b = w0b_ref[r]
        w0d = w0d_ref[r]
        b0 = b0_ref[r]
        w1 = w1_ref[r]
        b1 = b1_ref[r]
        nwh = nwh_ref[r]
        nwa = nwa_ref[r]
        nb0 = nb0_ref[r]
        nw1 = nw1_ref[r]
        nb1 = nb1_ref[r]

        P2 = jnp.dot(h2, w0a) + b0
        Q2 = jnp.dot(h2, w0b)
        h3 = h2.reshape(B, NP, HID)
        G3 = lax.dot_general(h3, h3, (((2,), (2,)), ((0,), (0,))),
                             precision=lax.Precision.HIGHEST)

        n2l = jnp.sum(G3 * eye[None], axis=2, keepdims=True)
        n2r = jnp.sum(G3 * eye[None], axis=1, keepdims=True)
        D3 = n2l + n2r - 2.0 * G3

        ppart = pltpu.repeat(P2.reshape(B, NP, EH), NP, axis=1)
        qhi = Q2.astype(jnp.bfloat16).astype(jnp.float32)
        qlo = (Q2 - qhi).reshape(B, NP, EH)
        q3 = qhi.reshape(B, NP, EH)
        qpart = jnp.concatenate(
            [jnp.dot(selj, q3[b]) + jnp.dot(selj, qlo[b])
             for b in range(B)], axis=0)
        rD = pltpu.repeat(D3, NP, axis=1)
        dcol = jnp.sum(rD * selj[None], axis=2,
                       keepdims=True).reshape(E, 1)

        dterm = (dcol.astype(jnp.bfloat16).astype(jnp.float32)
                 * w0d.astype(jnp.bfloat16).astype(jnp.float32))
        pre = ppart.reshape(E, EH) + qpart + dterm
        e2 = _lrelu(pre)
        f2 = _lrelu(jnp.dot(e2, w1) + b1)
        f4 = f2.reshape(B, NP, NP, EH)
        agg = jnp.sum(f4[:, :NN], axis=1).reshape(M, EH)
        n1 = _lrelu(jnp.dot(h2, nwh) + jnp.dot(agg, nwa) + nb0)
        h2 = _lrelu(jnp.dot(n1, nw1) + nb1)

    lat = jnp.dot(h2, wout_ref[...]) + bout_ref[...]
    out_ref[...] = lat.reshape(B, NP, LAT)[:, :NN, :]


def _mix_body(lat_ref, wm_ref, out_ref):
    out_ref[...] = jnp.dot(lat_ref[...], wm_ref[...])


def kernel(x, params):
    bs = x.shape[0]
    h0 = jnp.pad(x, ((0, 0), (0, NP - NN), (0, HID - IN)))

    w0a = jnp.stack([params['ew%d_0' % i][:, :HID].T for i in range(NMP)])
    w0b = jnp.stack([params['ew%d_0' % i][:, HID:2 * HID].T for i in range(NMP)])
    w0d = jnp.stack([params['ew%d_0' % i][:, 2 * HID:].T for i in range(NMP)])
    b0 = jnp.stack([params['eb%d_0' % i][None] for i in range(NMP)])
    w1 = jnp.stack([params['ew%d_1' % i].T for i in range(NMP)])
    b1 = jnp.stack([params['eb%d_1' % i][None] for i in range(NMP)])
    nwh = jnp.stack([params['nw%d_0' % i][:, :HID].T for i in range(NMP)])
    nwa = jnp.stack([params['nw%d_0' % i][:, HID:].T for i in range(NMP)])
    nb0 = jnp.stack([params['nb%d_0' % i][None] for i in range(NMP)])
    nw1 = jnp.stack([params['nw%d_1' % i].T for i in range(NMP)])
    nb1 = jnp.stack([params['nb%d_1' % i][None] for i in range(NMP)])
    wout = params['w_out'].T
    bout = params['b_out'][None]
    wmix = params['w_mix'].T

    full = lambda s: pl.BlockSpec(s, lambda i: (0,) * len(s))
    lat = pl.pallas_call(
        _encoder_body,
        grid=(bs // BB,),
        in_specs=[
            pl.BlockSpec((BB, NP, HID), lambda i: (i, 0, 0)),
            full((NMP, HID, EH)), full((NMP, HID, EH)), full((NMP, 1, EH)),
            full((NMP, 1, EH)), full((NMP, EH, EH)), full((NMP, 1, EH)),
            full((NMP, HID, HID)), full((NMP, EH, HID)), full((NMP, 1, HID)),
            full((NMP, HID, HID)), full((NMP, 1, HID)),
            full((HID, LAT)), full((1, LAT)),
        ],
        out_specs=pl.BlockSpec((BB, NN, LAT), lambda i: (i, 0, 0)),
        out_shape=jax.ShapeDtypeStruct((bs, NN, LAT), jnp.float32),
        compiler_params=pltpu.CompilerParams(
            dimension_semantics=("parallel",),
        ),
    )(h0, w0a, w0b, w0d, b0, w1, b1, nwh, nwa, nb0, nw1, nb1, wout, bout)

    z = pl.pallas_call(
        _mix_body,
        out_shape=jax.ShapeDtypeStruct((bs, LAT), jnp.float32),
    )(lat.reshape(bs, NN * LAT), wmix)
    return z[None]


# speedup vs baseline: 1.0489x; 1.0446x over previous
import jax
import jax.numpy as jnp
from jax import lax
from jax.experimental import pallas as pl
from jax.experimental.pallas import tpu as pltpu

NN = 30
NP = 32
IN = 4
HID = 64
EH = 96
LAT = 16
NMP = 3
ALPHA = 0.2

BB = 16


def _lrelu(v):
    return jnp.maximum(v, ALPHA * v)


def _encoder_body(x_ref, w0a_ref, w0b_ref, w0d_ref, b0_ref, w1_ref, b1_ref,
                  nwh_ref, nwa_ref, nb0_ref, nw1_ref, nb1_ref,
                  wout_ref, bout_ref, out_ref):
    B = x_ref.shape[0]
    M = B * NP
    E = B * NP * NP
    h2 = x_ref[...].reshape(M, HID)
    selj = (lax.broadcasted_iota(jnp.int32, (NP * NP, NP), 0) // NP
            == lax.broadcasted_iota(jnp.int32, (NP * NP, NP), 1)
            ).astype(jnp.float32)
    rows = lax.broadcasted_iota(jnp.int32, (NP * NP, 4 * NP), 0)
    cols = lax.broadcasted_iota(jnp.int32, (NP * NP, 4 * NP), 1)
    tgt = jnp.where(cols < 2 * NP, rows // NP, rows % NP)
    sel4 = ((cols % NP) == tgt).astype(jnp.float32)
    eye = (lax.broadcasted_iota(jnp.int32, (NP, NP), 0)
           == lax.broadcasted_iota(jnp.int32, (NP, NP), 1)
           ).astype(jnp.float32)
    for r in range(NMP):
        w0a = w0a_ref[r]
        w0b = w0b_ref[r]
        w0d = w0d_ref[r]
        b0 = b0_ref[r]
        w1 = w1_ref[r]
        b1 = b1_ref[r]
        nwh = nwh_ref[r]
        nwa = nwa_ref[r]
        nb0 = nb0_ref[r]
        nw1 = nw1_ref[r]
        nb1 = nb1_ref[r]

        P2 = jnp.dot(h2, w0a) + b0
        Q2 = jnp.dot(h2, w0b)
        h3 = h2.reshape(B, NP, HID)
        G3 = lax.dot_general(h3, h3, (((2,), (2,)), ((0,), (0,))),
                             precision=lax.Precision.HIGHEST)

        n2l = jnp.sum(G3 * eye[None], axis=2, keepdims=True)
        n2r = jnp.sum(G3 * eye[None], axis=1, keepdims=True)
        D3 = n2l + n2r - 2.0 * G3

        qhi = Q2.astype(jnp.bfloat16).astype(jnp.float32)
        qlo = (Q2 - qhi).reshape(B, NP, EH)
        q3 = qhi.reshape(B, NP, EH)
        phi = P2.astype(jnp.bfloat16).astype(jnp.float32)
        plo = (P2 - phi).reshape(B, NP, EH)
        p3 = phi.reshape(B, NP, EH)
        qpart = jnp.concatenate(
            [jnp.dot(sel4, jnp.concatenate(
                [q3[b], qlo[b], p3[b], plo[b]], axis=0))
             for b in range(B)], axis=0)
        rD = pltpu.repeat(D3, NP, axis=1)
        dcol = jnp.sum(rD * selj[None], axis=2,
                       keepdims=True).reshape(E, 1)

        dterm = (dcol.astype(jnp.bfloat16).astype(jnp.float32)
                 * w0d.astype(jnp.bfloat16).astype(jnp.float32))
        pre = qpart + dterm
        e2 = _lrelu(pre)
        f2 = _lrelu(jnp.dot(e2, w1) + b1)
        f4 = f2.reshape(B, NP, NP, EH)
        agg = jnp.sum(f4[:, :NN], axis=1).reshape(M, EH)
        n1 = _lrelu(jnp.dot(h2, nwh) + jnp.dot(agg, nwa) + nb0)
        h2 = _lrelu(jnp.dot(n1, nw1) + nb1)

    lat = jnp.dot(h2, wout_ref[...]) + bout_ref[...]
    out_ref[...] = lat.reshape(B, NP, LAT)[:, :NN, :]


def _mix_body(lat_ref, wm_ref, out_ref):
    out_ref[...] = jnp.dot(lat_ref[...], wm_ref[...])


def kernel(x, params):
    bs = x.shape[0]
    h0 = jnp.pad(x, ((0, 0), (0, NP - NN), (0, HID - IN)))

    w0a = jnp.stack([params['ew%d_0' % i][:, :HID].T for i in range(NMP)])
    w0b = jnp.stack([params['ew%d_0' % i][:, HID:2 * HID].T for i in range(NMP)])
    w0d = jnp.stack([params['ew%d_0' % i][:, 2 * HID:].T for i in range(NMP)])
    b0 = jnp.stack([params['eb%d_0' % i][None] for i in range(NMP)])
    w1 = jnp.stack([params['ew%d_1' % i].T for i in range(NMP)])
    b1 = jnp.stack([params['eb%d_1' % i][None] for i in range(NMP)])
    nwh = jnp.stack([params['nw%d_0' % i][:, :HID].T for i in range(NMP)])
    nwa = jnp.stack([params['nw%d_0' % i][:, HID:].T for i in range(NMP)])
    nb0 = jnp.stack([params['nb%d_0' % i][None] for i in range(NMP)])
    nw1 = jnp.stack([params['nw%d_1' % i].T for i in range(NMP)])
    nb1 = jnp.stack([params['nb%d_1' % i][None] for i in range(NMP)])
    wout = params['w_out'].T
    bout = params['b_out'][None]
    wmix = params['w_mix'].T

    full = lambda s: pl.BlockSpec(s, lambda i: (0,) * len(s))
    lat = pl.pallas_call(
        _encoder_body,
        grid=(bs // BB,),
        in_specs=[
            pl.BlockSpec((BB, NP, HID), lambda i: (i, 0, 0)),
            full((NMP, HID, EH)), full((NMP, HID, EH)), full((NMP, 1, EH)),
            full((NMP, 1, EH)), full((NMP, EH, EH)), full((NMP, 1, EH)),
            full((NMP, HID, HID)), full((NMP, EH, HID)), full((NMP, 1, HID)),
            full((NMP, HID, HID)), full((NMP, 1, HID)),
            full((HID, LAT)), full((1, LAT)),
        ],
        out_specs=pl.BlockSpec((BB, NN, LAT), lambda i: (i, 0, 0)),
        out_shape=jax.ShapeDtypeStruct((bs, NN, LAT), jnp.float32),
        compiler_params=pltpu.CompilerParams(
            dimension_semantics=("parallel",),
        ),
    )(h0, w0a, w0b, w0d, b0, w1, b1, nwh, nwa, nb0, nw1, nb1, wout, bout)

    z = pl.pallas_call(
        _mix_body,
        out_shape=jax.ShapeDtypeStruct((bs, LAT), jnp.float32),
    )(lat.reshape(bs, NN * LAT), wmix)
    return z[None]


# speedup vs baseline: 1.0861x; 1.0355x over previous
import jax
import jax.numpy as jnp
from jax import lax
from jax.experimental import pallas as pl
from jax.experimental.pallas import tpu as pltpu

NN = 30
NP = 32
IN = 4
HID = 64
EH = 96
LAT = 16
NMP = 3
ALPHA = 0.2

BB = 16


def _lrelu(v):
    return jnp.maximum(v, ALPHA * v)


def _encoder_body(x_ref, ew0_refs, eb0_refs, ew1_refs, eb1_refs,
                  nw0_refs, nb0_refs, nw1_refs, nb1_refs,
                  wout_ref, bout_ref, out_ref):
    B = x_ref.shape[0]
    M = B * NP
    E = B * NP * NP
    h2 = x_ref[...].reshape(M, HID)
    selj = (lax.broadcasted_iota(jnp.int32, (NP * NP, NP), 0) // NP
            == lax.broadcasted_iota(jnp.int32, (NP * NP, NP), 1)
            ).astype(jnp.float32)
    rows = lax.broadcasted_iota(jnp.int32, (NP * NP, 4 * NP), 0)
    cols = lax.broadcasted_iota(jnp.int32, (NP * NP, 4 * NP), 1)
    tgt = jnp.where(cols < 2 * NP, rows // NP, rows % NP)
    sel4 = ((cols % NP) == tgt).astype(jnp.float32)
    eye = (lax.broadcasted_iota(jnp.int32, (NP, NP), 0)
           == lax.broadcasted_iota(jnp.int32, (NP, NP), 1)
           ).astype(jnp.float32)
    for r in range(NMP):
        ew0 = ew0_refs[r][...]
        w0a = jnp.swapaxes(ew0[:, :HID], 0, 1)
        w0b = jnp.swapaxes(ew0[:, HID:2 * HID], 0, 1)
        w0d = jnp.swapaxes(ew0[:, 2 * HID:], 0, 1)
        b0 = eb0_refs[r][...]
        w1 = jnp.swapaxes(ew1_refs[r][...], 0, 1)
        b1 = eb1_refs[r][...]
        nw0 = nw0_refs[r][...]
        nwh = jnp.swapaxes(nw0[:, :HID], 0, 1)
        nwa = jnp.swapaxes(nw0[:, HID:], 0, 1)
        nb0 = nb0_refs[r][...]
        nw1 = jnp.swapaxes(nw1_refs[r][...], 0, 1)
        nb1 = nb1_refs[r][...]

        P2 = jnp.dot(h2, w0a) + b0
        Q2 = jnp.dot(h2, w0b)
        h3 = h2.reshape(B, NP, HID)
        G3 = lax.dot_general(h3, h3, (((2,), (2,)), ((0,), (0,))),
                             precision=lax.Precision.HIGHEST)

        n2l = jnp.sum(G3 * eye[None], axis=2, keepdims=True)
        n2r = jnp.sum(G3 * eye[None], axis=1, keepdims=True)
        D3 = n2l + n2r - 2.0 * G3

        qhi = Q2.astype(jnp.bfloat16).astype(jnp.float32)
        qlo = (Q2 - qhi).reshape(B, NP, EH)
        q3 = qhi.reshape(B, NP, EH)
        phi = P2.astype(jnp.bfloat16).astype(jnp.float32)
        plo = (P2 - phi).reshape(B, NP, EH)
        p3 = phi.reshape(B, NP, EH)
        qpart = jnp.concatenate(
            [jnp.dot(sel4, jnp.concatenate(
                [q3[b], qlo[b], p3[b], plo[b]], axis=0))
             for b in range(B)], axis=0)
        rD = pltpu.repeat(D3, NP, axis=1)
        dcol = jnp.sum(rD * selj[None], axis=2,
                       keepdims=True).reshape(E, 1)

        dterm = (dcol.astype(jnp.bfloat16).astype(jnp.float32)
                 * w0d.astype(jnp.bfloat16).astype(jnp.float32))
        pre = qpart + dterm
        e2 = _lrelu(pre)
        f2 = _lrelu(jnp.dot(e2, w1) + b1)
        f4 = f2.reshape(B, NP, NP, EH)
        agg = jnp.sum(f4[:, :NN], axis=1).reshape(M, EH)
        n1 = _lrelu(jnp.dot(h2, nwh) + jnp.dot(agg, nwa) + nb0)
        h2 = _lrelu(jnp.dot(n1, nw1) + nb1)

    lat = (jnp.dot(h2, jnp.swapaxes(wout_ref[...], 0, 1))
           + bout_ref[...])
    out_ref[...] = lat.reshape(B, NP, LAT)[:, :NN, :]


def _mix_body(lat_ref, wm_ref, out_ref):
    out_ref[...] = jnp.dot(lat_ref[...], jnp.swapaxes(wm_ref[...], 0, 1))


def kernel(x, params):
    bs = x.shape[0]
    h0 = jnp.pad(x, ((0, 0), (0, NP - NN), (0, HID - IN)))

    ew0 = [params['ew%d_0' % i] for i in range(NMP)]
    eb0 = [params['eb%d_0' % i][None] for i in range(NMP)]
    ew1 = [params['ew%d_1' % i] for i in range(NMP)]
    eb1 = [params['eb%d_1' % i][None] for i in range(NMP)]
    nw0 = [params['nw%d_0' % i] for i in range(NMP)]
    nb0 = [params['nb%d_0' % i][None] for i in range(NMP)]
    nw1 = [params['nw%d_1' % i] for i in range(NMP)]
    nb1 = [params['nb%d_1' % i][None] for i in range(NMP)]
    wargs = (*ew0, *eb0, *ew1, *eb1, *nw0, *nb0, *nw1, *nb1,
             params['w_out'], params['b_out'][None])

    def body(x_r, *rest):
        _encoder_body(x_r, rest[0:3], rest[3:6], rest[6:9], rest[9:12],
                      rest[12:15], rest[15:18], rest[18:21], rest[21:24],
                      rest[24], rest[25], rest[26])

    full = lambda a: pl.BlockSpec(a.shape, lambda i: (0,) * a.ndim)
    lat = pl.pallas_call(
        body,
        grid=(bs // BB,),
        in_specs=[pl.BlockSpec((BB, NP, HID), lambda i: (i, 0, 0))]
                 + [full(a) for a in wargs],
        out_specs=pl.BlockSpec((BB, NN, LAT), lambda i: (i, 0, 0)),
        out_shape=jax.ShapeDtypeStruct((bs, NN, LAT), jnp.float32),
        compiler_params=pltpu.CompilerParams(
            dimension_semantics=("parallel",),
        ),
    )(h0, *wargs)

    z = pl.pallas_call(
        _mix_body,
        out_shape=jax.ShapeDtypeStruct((bs, LAT), jnp.float32),
    )(lat.reshape(bs, NN * LAT), params['w_mix'])
    return z[None]
